# baseline (speedup 1.0000x reference)
import functools

import jax
import jax.numpy as jnp
from jax import lax
from jax.experimental import pallas as pl
from jax.experimental.pallas import tpu as pltpu
from jax.experimental.pallas import tpu_sc as plsc

NC = 2
NS = 16
NW = NC * NS
D = 64
C = 128
NB = 16384


def _repack_body(xa_ref, xb_ref, o_ref):
    o_ref[:, 0:64] = xa_ref[...].T
    o_ref[:, 64:128] = xb_ref[...].T


def _repack(wT, V):
    ngrid = pl.cdiv(V, 2 * NB)
    last = V // NB
    return pl.pallas_call(
        _repack_body,
        out_shape=jax.ShapeDtypeStruct((ngrid * NB, 128), jnp.float32),
        grid=(ngrid,),
        in_specs=[
            pl.BlockSpec((64, NB), lambda j: (0, jnp.minimum(2 * j, last))),
            pl.BlockSpec((64, NB), lambda j: (0, jnp.minimum(2 * j + 1, last))),
        ],
        out_specs=pl.BlockSpec((NB, 128), lambda j: (j, 0)),
    )(wT, wT)


def _remap(r):
    j = r // (2 * NB)
    c = r % (2 * NB)
    return 2 * (j * NB + c % NB) + c // NB


@functools.lru_cache(maxsize=None)
def _gather_t(S1):
    mesh = plsc.VectorSubcoreMesh(core_axis_name="c", subcore_axis_name="s")

    @functools.partial(
        pl.kernel,
        out_type=jax.ShapeDtypeStruct((S1, 8, NW, 8, 128), jnp.float32),
        mesh=mesh,
        compiler_params=pltpu.CompilerParams(
            use_tc_tiling_on_sc=False, needs_layout_passes=False),
        scratch_types=[
            pltpu.VMEM((S1, C), jnp.int32),
            pltpu.VMEM((2, C, D), jnp.float32),
            pltpu.VMEM((2, 8, 8, 128), jnp.float32),
            pltpu.SemaphoreType.DMA((2,)),
            pltpu.SemaphoreType.DMA((2,)),
        ],
    )
    def k(ids_hbm, w_hbm, out_hbm, idsw, gbuf, tbuf, gsem, osem):
        wid = lax.axis_index("s") * NC + lax.axis_index("c")
        pltpu.sync_copy(ids_hbm.at[wid], idsw)

        def fire(s, b):
            pltpu.async_copy(w_hbm.at[idsw.at[s]], gbuf.at[b], gsem.at[b])

        def wait_g(b):
            pltpu.make_async_copy(
                w_hbm.at[pl.ds(0, C)], gbuf.at[b], gsem.at[b]
            ).wait()

        def start_o(s, b):
            pltpu.async_copy(tbuf.at[b], out_hbm.at[s, :, wid], osem.at[b])

        def wait_o(b):
            pltpu.make_async_copy(
                tbuf.at[b], out_hbm.at[0, :, 0], osem.at[b]
            ).wait()

        def transpose(b):
            for g in range(8):
                rows = lax.broadcasted_iota(jnp.int32, (16,), 0) + 16 * g
                for i in range(8):
                    for dl in range(8):
                        cols = jnp.full((16,), 8 * i + dl, jnp.int32)
                        v = plsc.load_gather(gbuf.at[b], [rows, cols])
                        tbuf[b, i, dl, pl.ds(16 * g, 16)] = v

        fire(0, 0)

        @pl.loop(0, S1 // 2)
        def _pair(t):
            s0 = 2 * t
            for b in (0, 1):
                s = s0 + b
                nb = 1 - b

                @pl.when(s + 1 < S1)
                def _fire_next():
                    fire(s + 1, nb)

                wait_g(b)

                @pl.when(s >= 2)
                def _recycle():
                    wait_o(b)

                transpose(b)
                start_o(s, b)

        wait_o(0)
        wait_o(1)

    return k


@jax.jit
def kernel(ids, weight):
    S0, S1 = ids.shape
    ids_r = _remap(ids.astype(jnp.int32))
    ids_t = ids_r.T.reshape(S1, NW, C).transpose(1, 0, 2)
    V = weight.shape[0]
    w2 = _repack(weight.T, V)
    w_lin = w2.reshape(w2.shape[0] * 2, D)
    out5 = _gather_t(S1)(ids_t, w_lin)
    return out5.transpose(2, 4, 0, 1, 3).reshape(S0, S1, D)
